# Initial kernel scaffold; baseline (speedup 1.0000x reference)
import functools

import jax
import jax.numpy as jnp
from jax import lax
from jax.experimental import pallas as pl
from jax.experimental.pallas import tpu as pltpu
from jax.experimental.pallas import tpu_sc as plsc

_N = 10000
_E = 320000
_D = 128
_G = 16
_DOUT = 64
_NC = 2
_NS = 16
_NW = _NC * _NS
_K = 128
_NCHUNK = 80
_EP = _NW * _NCHUNK * _K
_NACC = 10016
_ZROWS = _NACC // _NS
_WROWS = _N // _NS
_BLK = 2000

_sc_mesh = plsc.VectorSubcoreMesh(core_axis_name="c", subcore_axis_name="s")


@functools.partial(
    pl.kernel,
    out_type=jax.ShapeDtypeStruct((_NC, _N, 16), jnp.float32),
    mesh=_sc_mesh,
    scratch_types=[
        pltpu.VMEM((_NCHUNK, _K), jnp.int32),
        pltpu.VMEM((_K, 16), jnp.float32),
        pltpu.VMEM_SHARED((_NACC, 16), jnp.float32),
    ],
)
def _deg_kernel(dstp_hbm, ones_hbm, zdeg_hbm, out_hbm, didx, ones_v, acc):
    c = lax.axis_index("c")
    s = lax.axis_index("s")
    w = c * _NS + s
    pltpu.sync_copy(zdeg_hbm, acc.at[pl.ds(s * _ZROWS, _ZROWS)])
    pltpu.sync_copy(ones_hbm, ones_v)
    pltpu.sync_copy(dstp_hbm.at[w], didx)
    plsc.subcore_barrier()

    def step(j, carry):
        pltpu.sync_copy(ones_v, acc.at[didx.at[j]], add=True)
        return carry

    lax.fori_loop(0, _NCHUNK, step, 0)
    plsc.subcore_barrier()
    pltpu.sync_copy(acc.at[pl.ds(s * _WROWS, _WROWS)],
                    out_hbm.at[c, pl.ds(s * _WROWS, _WROWS)])


@functools.partial(
    pl.kernel,
    out_type=jax.ShapeDtypeStruct((_NC, _N, _D), jnp.float32),
    mesh=_sc_mesh,
    scratch_types=[
        pltpu.VMEM((_NCHUNK, _K), jnp.int32),
        pltpu.VMEM((_NCHUNK, _K), jnp.int32),
        pltpu.VMEM((_K, _D), jnp.float32),
        pltpu.VMEM_SHARED((_NACC, _D), jnp.float32),
    ],
)
def _agg_kernel(g_hbm, srcp_hbm, dstp_hbm, zrow_hbm, out_hbm, sidx, didx, buf, acc):
    c = lax.axis_index("c")
    s = lax.axis_index("s")
    w = c * _NS + s
    pltpu.sync_copy(zrow_hbm, acc.at[pl.ds(s * _ZROWS, _ZROWS)])
    pltpu.sync_copy(srcp_hbm.at[w], sidx)
    pltpu.sync_copy(dstp_hbm.at[w], didx)
    plsc.subcore_barrier()

    def step(j, carry):
        pltpu.sync_copy(g_hbm.at[sidx.at[j]], buf)
        pltpu.sync_copy(buf, acc.at[didx.at[j]], add=True)
        return carry

    lax.fori_loop(0, _NCHUNK, step, 0)
    plsc.subcore_barrier()
    pltpu.sync_copy(acc.at[pl.ds(s * _WROWS, _WROWS)],
                    out_hbm.at[c, pl.ds(s * _WROWS, _WROWS)])


def _dinv_from(degp):
    deg = degp[0, :, 0:1] + degp[1, :, 0:1] + 1.0
    return lax.rsqrt(deg)


def _tc_scale_body(x_ref, w_ref, degp_ref, o_ref):
    dinv = _dinv_from(degp_ref[...])
    h = jnp.dot(x_ref[...], w_ref[...], preferred_element_type=jnp.float32)
    o_ref[...] = h * dinv


def _tc_mid_body(sp_ref, g_ref, degp_ref, b_ref, w_ref, o_ref):
    dinv = _dinv_from(degp_ref[...])
    ssum = sp_ref[0] + sp_ref[1] + g_ref[...]
    h = jnp.maximum(ssum * dinv + b_ref[...], 0.0)
    o_ref[...] = jnp.dot(h, w_ref[...], preferred_element_type=jnp.float32) * dinv


def _tc_pool_body(sp_ref, g_ref, degp_ref, b_ref, batch_ref, wfc_ref, bfc_ref,
                  o_ref, sums_ref, cnt_ref):
    i = pl.program_id(0)
    dinv = _dinv_from(degp_ref[...])
    h = jnp.maximum((sp_ref[0] + sp_ref[1] + g_ref[...]) * dinv + b_ref[...], 0.0)
    gid = lax.broadcasted_iota(jnp.float32, (_BLK, _G), 1)
    onehot = (batch_ref[...] == gid).astype(jnp.float32)
    dn = (((0,), (0,)), ((), ()))
    ps = lax.dot_general(onehot, h, dn, preferred_element_type=jnp.float32)
    pc = lax.dot_general(onehot, jnp.ones((_BLK, _D), jnp.float32), dn,
                         preferred_element_type=jnp.float32)

    @pl.when(i == 0)
    def _():
        sums_ref[...] = jnp.zeros_like(sums_ref)
        cnt_ref[...] = jnp.zeros_like(cnt_ref)

    sums_ref[...] += ps
    cnt_ref[...] += pc
    pooled = sums_ref[...] / jnp.maximum(cnt_ref[...], 1.0)
    o_ref[...] = jnp.dot(pooled, wfc_ref[...],
                         preferred_element_type=jnp.float32) + bfc_ref[...]


def _tc_scale(x, W, degp):
    return pl.pallas_call(
        _tc_scale_body,
        grid=(_N // _BLK,),
        in_specs=[
            pl.BlockSpec((_BLK, _D), lambda i: (i, 0)),
            pl.BlockSpec((_D, _D), lambda i: (0, 0)),
            pl.BlockSpec((2, _BLK, 16), lambda i: (0, i, 0)),
        ],
        out_specs=pl.BlockSpec((_BLK, _D), lambda i: (i, 0)),
        out_shape=jax.ShapeDtypeStruct((_N, _D), jnp.float32),
    )(x, W, degp)


def _tc_mid(sp, g, degp, b, W):
    return pl.pallas_call(
        _tc_mid_body,
        grid=(_N // _BLK,),
        in_specs=[
            pl.BlockSpec((2, _BLK, _D), lambda i: (0, i, 0)),
            pl.BlockSpec((_BLK, _D), lambda i: (i, 0)),
            pl.BlockSpec((2, _BLK, 16), lambda i: (0, i, 0)),
            pl.BlockSpec((1, _D), lambda i: (0, 0)),
            pl.BlockSpec((_D, _D), lambda i: (0, 0)),
        ],
        out_specs=pl.BlockSpec((_BLK, _D), lambda i: (i, 0)),
        out_shape=jax.ShapeDtypeStruct((_N, _D), jnp.float32),
    )(sp, g, degp, b, W)


def _tc_pool(sp, g, degp, b, batchf, Wfc, bfc):
    return pl.pallas_call(
        _tc_pool_body,
        grid=(_N // _BLK,),
        in_specs=[
            pl.BlockSpec((2, _BLK, _D), lambda i: (0, i, 0)),
            pl.BlockSpec((_BLK, _D), lambda i: (i, 0)),
            pl.BlockSpec((2, _BLK, 16), lambda i: (0, i, 0)),
            pl.BlockSpec((1, _D), lambda i: (0, 0)),
            pl.BlockSpec((_BLK, 1), lambda i: (i, 0)),
            pl.BlockSpec((_D, _DOUT), lambda i: (0, 0)),
            pl.BlockSpec((1, _DOUT), lambda i: (0, 0)),
        ],
        out_specs=pl.BlockSpec((_G, _DOUT), lambda i: (0, 0)),
        out_shape=jax.ShapeDtypeStruct((_G, _DOUT), jnp.float32),
        scratch_shapes=[
            pltpu.VMEM((_G, _D), jnp.float32),
            pltpu.VMEM((_G, _D), jnp.float32),
        ],
    )(sp, g, degp, b, batchf, Wfc, bfc)


def kernel(x, edge_index, batch, W1, b1, W2, b2, Wfc, bfc):
    src = edge_index[0].astype(jnp.int32)
    dst = edge_index[1].astype(jnp.int32)
    pad = _EP - _E
    srcp = jnp.concatenate([src, jnp.zeros((pad,), jnp.int32)]).reshape(_NW, _NCHUNK, _K)
    dstp = jnp.concatenate([dst, jnp.full((pad,), _N, jnp.int32)]).reshape(_NW, _NCHUNK, _K)

    ones16 = jnp.ones((_K, 16), jnp.float32)
    zdeg = jnp.zeros((_ZROWS, 16), jnp.float32)
    zrow = jnp.zeros((_ZROWS, _D), jnp.float32)

    degp = _deg_kernel(dstp, ones16, zdeg)

    g1 = _tc_scale(x, W1, degp)
    s1 = _agg_kernel(g1, srcp, dstp, zrow)
    g2 = _tc_mid(s1, g1, degp, b1.reshape(1, _D), W2)
    s2 = _agg_kernel(g2, srcp, dstp, zrow)

    batchf = batch.astype(jnp.float32).reshape(_N, 1)
    return _tc_pool(s2, g2, degp, b2.reshape(1, _D),
                    batchf, Wfc, bfc.reshape(1, _DOUT))


# speedup vs baseline: 8.3331x; 8.3331x over previous
import functools

import jax
import jax.numpy as jnp
from jax import lax
from jax.experimental import pallas as pl
from jax.experimental.pallas import tpu as pltpu
from jax.experimental.pallas import tpu_sc as plsc

_N = 10000
_E = 320000
_D = 128
_G = 16
_DOUT = 64
_NC = 2
_NS = 16
_NW = _NC * _NS
_K = 128
_NCHUNK = 80
_EP = _NW * _NCHUNK * _K
_NACC = 10112
_ZROWS = _NACC // _NS
_WROWS = 624
_TAIL = _N - _NS * _WROWS
_BLK = 2000

_sc_mesh = plsc.VectorSubcoreMesh(core_axis_name="c", subcore_axis_name="s")


@functools.partial(
    pl.kernel,
    out_type=jax.ShapeDtypeStruct((_NC, _N, _D), jnp.float32),
    mesh=_sc_mesh,
    scratch_types=[
        pltpu.VMEM((_NCHUNK, _K), jnp.int32),
        pltpu.VMEM((_K, _D), jnp.float32),
        pltpu.VMEM_SHARED((_NACC, _D), jnp.float32),
    ],
)
def _deg_kernel(dstp_hbm, ones_hbm, zdeg_hbm, out_hbm, didx, ones_v, acc):
    c = lax.axis_index("c")
    s = lax.axis_index("s")
    w = c * _NS + s
    pltpu.sync_copy(zdeg_hbm, acc.at[pl.ds(s * _ZROWS, _ZROWS)])
    pltpu.sync_copy(ones_hbm, ones_v)
    pltpu.sync_copy(dstp_hbm.at[w], didx)
    plsc.subcore_barrier()

    def step(j, carry):
        pltpu.sync_copy(ones_v, acc.at[didx.at[j]], add=True)
        return carry

    lax.fori_loop(0, _NCHUNK, step, 0)
    plsc.subcore_barrier()
    pltpu.sync_copy(acc.at[pl.ds(s * _WROWS, _WROWS)],
                    out_hbm.at[c, pl.ds(s * _WROWS, _WROWS)])

    @pl.when(s == _NS - 1)
    def _():
        pltpu.sync_copy(acc.at[pl.ds(_NS * _WROWS, _TAIL)],
                        out_hbm.at[c, pl.ds(_NS * _WROWS, _TAIL)])


@functools.partial(
    pl.kernel,
    out_type=jax.ShapeDtypeStruct((_NC, _N, _D), jnp.float32),
    mesh=_sc_mesh,
    scratch_types=[
        pltpu.VMEM((_NCHUNK, _K), jnp.int32),
        pltpu.VMEM((_NCHUNK, _K), jnp.int32),
        pltpu.VMEM((_K, _D), jnp.float32),
        pltpu.VMEM_SHARED((_NACC, _D), jnp.float32),
    ],
)
def _agg_kernel(g_hbm, srcp_hbm, dstp_hbm, zrow_hbm, out_hbm, sidx, didx, buf, acc):
    c = lax.axis_index("c")
    s = lax.axis_index("s")
    w = c * _NS + s
    pltpu.sync_copy(zrow_hbm, acc.at[pl.ds(s * _ZROWS, _ZROWS)])
    pltpu.sync_copy(srcp_hbm.at[w], sidx)
    pltpu.sync_copy(dstp_hbm.at[w], didx)
    plsc.subcore_barrier()

    def step(j, carry):
        pltpu.sync_copy(g_hbm.at[sidx.at[j]], buf)
        pltpu.sync_copy(buf, acc.at[didx.at[j]], add=True)
        return carry

    lax.fori_loop(0, _NCHUNK, step, 0)
    plsc.subcore_barrier()
    pltpu.sync_copy(acc.at[pl.ds(s * _WROWS, _WROWS)],
                    out_hbm.at[c, pl.ds(s * _WROWS, _WROWS)])

    @pl.when(s == _NS - 1)
    def _():
        pltpu.sync_copy(acc.at[pl.ds(_NS * _WROWS, _TAIL)],
                        out_hbm.at[c, pl.ds(_NS * _WROWS, _TAIL)])


def _dinv_from(degp):
    deg = degp[0, :, 0:1] + degp[1, :, 0:1] + 1.0
    return lax.rsqrt(deg)


def _tc_scale_body(x_ref, w_ref, degp_ref, o_ref):
    dinv = _dinv_from(degp_ref[...])
    h = jnp.dot(x_ref[...], w_ref[...], preferred_element_type=jnp.float32)
    o_ref[...] = h * dinv


def _tc_mid_body(sp_ref, g_ref, degp_ref, b_ref, w_ref, o_ref):
    dinv = _dinv_from(degp_ref[...])
    ssum = sp_ref[0] + sp_ref[1] + g_ref[...]
    h = jnp.maximum(ssum * dinv + b_ref[...], 0.0)
    o_ref[...] = jnp.dot(h, w_ref[...], preferred_element_type=jnp.float32) * dinv


def _tc_pool_body(sp_ref, g_ref, degp_ref, b_ref, batch_ref, wfc_ref, bfc_ref,
                  o_ref, sums_ref, cnt_ref):
    i = pl.program_id(0)
    dinv = _dinv_from(degp_ref[...])
    h = jnp.maximum((sp_ref[0] + sp_ref[1] + g_ref[...]) * dinv + b_ref[...], 0.0)
    gid = lax.broadcasted_iota(jnp.int32, (_BLK, _G), 1)
    onehot = (batch_ref[...] == gid).astype(jnp.float32)
    dn = (((0,), (0,)), ((), ()))
    ps = lax.dot_general(onehot, h, dn, preferred_element_type=jnp.float32)
    pc = lax.dot_general(onehot, jnp.ones((_BLK, _D), jnp.float32), dn,
                         preferred_element_type=jnp.float32)

    @pl.when(i == 0)
    def _():
        sums_ref[...] = jnp.zeros_like(sums_ref)
        cnt_ref[...] = jnp.zeros_like(cnt_ref)

    sums_ref[...] += ps
    cnt_ref[...] += pc
    pooled = sums_ref[...] / jnp.maximum(cnt_ref[...], 1.0)
    o_ref[...] = jnp.dot(pooled, wfc_ref[...],
                         preferred_element_type=jnp.float32) + bfc_ref[...]


def _tc_scale(x, W, degp):
    return pl.pallas_call(
        _tc_scale_body,
        grid=(_N // _BLK,),
        in_specs=[
            pl.BlockSpec((_BLK, _D), lambda i: (i, 0)),
            pl.BlockSpec((_D, _D), lambda i: (0, 0)),
            pl.BlockSpec((2, _BLK, _D), lambda i: (0, i, 0)),
        ],
        out_specs=pl.BlockSpec((_BLK, _D), lambda i: (i, 0)),
        out_shape=jax.ShapeDtypeStruct((_N, _D), jnp.float32),
    )(x, W, degp)


def _tc_mid(sp, g, degp, b, W):
    return pl.pallas_call(
        _tc_mid_body,
        grid=(_N // _BLK,),
        in_specs=[
            pl.BlockSpec((2, _BLK, _D), lambda i: (0, i, 0)),
            pl.BlockSpec((_BLK, _D), lambda i: (i, 0)),
            pl.BlockSpec((2, _BLK, _D), lambda i: (0, i, 0)),
            pl.BlockSpec((1, _D), lambda i: (0, 0)),
            pl.BlockSpec((_D, _D), lambda i: (0, 0)),
        ],
        out_specs=pl.BlockSpec((_BLK, _D), lambda i: (i, 0)),
        out_shape=jax.ShapeDtypeStruct((_N, _D), jnp.float32),
    )(sp, g, degp, b, W)


def _tc_pool(sp, g, degp, b, batchf, Wfc, bfc):
    return pl.pallas_call(
        _tc_pool_body,
        grid=(_N // _BLK,),
        in_specs=[
            pl.BlockSpec((2, _BLK, _D), lambda i: (0, i, 0)),
            pl.BlockSpec((_BLK, _D), lambda i: (i, 0)),
            pl.BlockSpec((2, _BLK, _D), lambda i: (0, i, 0)),
            pl.BlockSpec((1, _D), lambda i: (0, 0)),
            pl.BlockSpec((_BLK, 1), lambda i: (i, 0)),
            pl.BlockSpec((_D, _DOUT), lambda i: (0, 0)),
            pl.BlockSpec((1, _DOUT), lambda i: (0, 0)),
        ],
        out_specs=pl.BlockSpec((_G, _DOUT), lambda i: (0, 0)),
        out_shape=jax.ShapeDtypeStruct((_G, _DOUT), jnp.float32),
        scratch_shapes=[
            pltpu.VMEM((_G, _D), jnp.float32),
            pltpu.VMEM((_G, _D), jnp.float32),
        ],
    )(sp, g, degp, b, batchf, Wfc, bfc)


def kernel(x, edge_index, batch, W1, b1, W2, b2, Wfc, bfc):
    src = edge_index[0].astype(jnp.int32)
    dst = edge_index[1].astype(jnp.int32)
    pad = _EP - _E
    srcp = jnp.concatenate([src, jnp.zeros((pad,), jnp.int32)]).reshape(_NW, _NCHUNK, _K)
    dstp = jnp.concatenate([dst, jnp.full((pad,), _N, jnp.int32)]).reshape(_NW, _NCHUNK, _K)

    ones = jnp.ones((_K, _D), jnp.float32)
    zrow = jnp.zeros((_ZROWS, _D), jnp.float32)

    degp = _deg_kernel(dstp, ones, zrow)

    g1 = _tc_scale(x, W1, degp)
    s1 = _agg_kernel(g1, srcp, dstp, zrow)
    g2 = _tc_mid(s1, g1, degp, b1.reshape(1, _D), W2)
    s2 = _agg_kernel(g2, srcp, dstp, zrow)

    batchf = batch.astype(jnp.int32).reshape(_N, 1)
    return _tc_pool(s2, g2, degp, b2.reshape(1, _D),
                    batchf, Wfc, bfc.reshape(1, _DOUT))
